# baseline (speedup 1.0000x reference)
import functools

import jax
import jax.numpy as jnp
from jax import lax
from jax.experimental import pallas as pl
from jax.experimental.pallas import tpu as pltpu
from jax.experimental.pallas import tpu_sc as plsc

VOCAB = 100000
NEMBED = 128
BATCH = 4096
CTX = 20


_NC = 2
_NS = 16
_NW = _NC * _NS
_BPW = BATCH // _NW
_CHUNK = 4
_NCHUNK = _BPW // _CHUNK
_LANES = NEMBED // 16

_sc_mesh = plsc.VectorSubcoreMesh(core_axis_name="c", subcore_axis_name="s")


@functools.partial(
    pl.kernel,
    mesh=_sc_mesh,
    out_type=jax.ShapeDtypeStruct((BATCH, NEMBED), jnp.float32),
    scratch_types=[
        pltpu.VMEM((_BPW * CTX,), jnp.int32),
        pltpu.VMEM((_CHUNK * CTX, NEMBED), jnp.float32),
        pltpu.VMEM((_BPW, NEMBED), jnp.float32),
        pltpu.SemaphoreType.DMA,
    ],
)
def _gather_mean(idx_hbm, table_hbm, out_hbm, idx_v, rows_v, acc_v, sem):
    wid = lax.axis_index("s") * _NC + lax.axis_index("c")
    base = wid * _BPW
    pltpu.sync_copy(idx_hbm.at[pl.ds(base * CTX, _BPW * CTX)], idx_v)

    def chunk_body(ci, carry):
        pltpu.async_copy(
            table_hbm.at[idx_v.at[pl.ds(ci * _CHUNK * CTX, _CHUNK * CTX)]],
            rows_v,
            sem,
        ).wait()
        for r in range(_CHUNK):
            for d in range(_LANES):
                acc = rows_v[r * CTX, pl.ds(d * 16, 16)]
                for c in range(1, CTX):
                    acc = acc + rows_v[r * CTX + c, pl.ds(d * 16, 16)]
                acc_v[ci * _CHUNK + r, pl.ds(d * 16, 16)] = acc * (1.0 / CTX)
        return carry

    lax.fori_loop(0, _NCHUNK, chunk_body, 0)
    pltpu.sync_copy(acc_v, out_hbm.at[pl.ds(base, _BPW)])


_BB = 1024
_VB = 2048


def _mm_body(x_ref, w_ref, b_ref, o_ref):
    x = x_ref[...].astype(jnp.bfloat16)
    w = w_ref[...].astype(jnp.bfloat16)
    acc = lax.dot_general(
        x, w, (((1,), (1,)), ((), ())), preferred_element_type=jnp.float32
    )
    o_ref[...] = acc + b_ref[...]


_matmul = pl.pallas_call(
    _mm_body,
    grid=(pl.cdiv(VOCAB, _VB), BATCH // _BB),
    in_specs=[
        pl.BlockSpec((_BB, NEMBED), lambda v, i: (i, 0)),
        pl.BlockSpec((_VB, NEMBED), lambda v, i: (v, 0)),
        pl.BlockSpec((1, _VB), lambda v, i: (0, v)),
    ],
    out_specs=pl.BlockSpec((_BB, _VB), lambda v, i: (i, v)),
    out_shape=jax.ShapeDtypeStruct((BATCH, VOCAB), jnp.float32),
)


def kernel(inp, embed_table, W, b):
    idx = inp.reshape(-1).astype(jnp.int32)
    avg = _gather_mean(idx, embed_table)
    return _matmul(avg, W, b.reshape(1, VOCAB))


# speedup vs baseline: 1.0053x; 1.0053x over previous
import functools

import jax
import jax.numpy as jnp
from jax import lax
from jax.experimental import pallas as pl
from jax.experimental.pallas import tpu as pltpu
from jax.experimental.pallas import tpu_sc as plsc

VOCAB = 100000
NEMBED = 128
BATCH = 4096
CTX = 20


_NC = 2
_NS = 16
_NW = _NC * _NS
_BPW = BATCH // _NW
_CHUNK = 4
_NCHUNK = _BPW // _CHUNK
_LANES = NEMBED // 16

_sc_mesh = plsc.VectorSubcoreMesh(core_axis_name="c", subcore_axis_name="s")


@functools.partial(
    pl.kernel,
    mesh=_sc_mesh,
    out_type=jax.ShapeDtypeStruct((BATCH, NEMBED), jnp.float32),
    scratch_types=[
        pltpu.VMEM((_BPW * CTX,), jnp.int32),
        pltpu.VMEM((_CHUNK * CTX, NEMBED), jnp.float32),
        pltpu.VMEM((_BPW, NEMBED), jnp.float32),
        pltpu.SemaphoreType.DMA,
    ],
)
def _gather_mean(idx_hbm, table_hbm, out_hbm, idx_v, rows_v, acc_v, sem):
    wid = lax.axis_index("s") * _NC + lax.axis_index("c")
    base = wid * _BPW
    pltpu.sync_copy(idx_hbm.at[pl.ds(base * CTX, _BPW * CTX)], idx_v)

    def chunk_body(ci, carry):
        pltpu.async_copy(
            table_hbm.at[idx_v.at[pl.ds(ci * _CHUNK * CTX, _CHUNK * CTX)]],
            rows_v,
            sem,
        ).wait()
        for r in range(_CHUNK):
            for d in range(_LANES):
                acc = rows_v[r * CTX, pl.ds(d * 16, 16)]
                for c in range(1, CTX):
                    acc = acc + rows_v[r * CTX + c, pl.ds(d * 16, 16)]
                acc_v[ci * _CHUNK + r, pl.ds(d * 16, 16)] = acc * (1.0 / CTX)
        return carry

    lax.fori_loop(0, _NCHUNK, chunk_body, 0)
    pltpu.sync_copy(acc_v, out_hbm.at[pl.ds(base, _BPW)])


_BB = 2048
_VB = 2048


def _mm_body(x_ref, w_ref, b_ref, o_ref):
    x = x_ref[...].astype(jnp.bfloat16)
    w = w_ref[...].astype(jnp.bfloat16)
    acc = lax.dot_general(
        x, w, (((1,), (1,)), ((), ())), preferred_element_type=jnp.float32
    )
    o_ref[...] = acc + b_ref[...]


_matmul = pl.pallas_call(
    _mm_body,
    grid=(pl.cdiv(VOCAB, _VB), BATCH // _BB),
    in_specs=[
        pl.BlockSpec((_BB, NEMBED), lambda v, i: (i, 0)),
        pl.BlockSpec((_VB, NEMBED), lambda v, i: (v, 0)),
        pl.BlockSpec((1, _VB), lambda v, i: (0, v)),
    ],
    out_specs=pl.BlockSpec((_BB, _VB), lambda v, i: (i, v)),
    out_shape=jax.ShapeDtypeStruct((BATCH, VOCAB), jnp.float32),
    compiler_params=pltpu.CompilerParams(
        dimension_semantics=("parallel", "parallel"),
    ),
)


def kernel(inp, embed_table, W, b):
    idx = inp.reshape(-1).astype(jnp.int32)
    avg = _gather_mean(idx, embed_table)
    return _matmul(avg, W, b.reshape(1, VOCAB))


# speedup vs baseline: 1.0130x; 1.0076x over previous
import functools

import jax
import jax.numpy as jnp
from jax import lax
from jax.experimental import pallas as pl
from jax.experimental.pallas import tpu as pltpu
from jax.experimental.pallas import tpu_sc as plsc

VOCAB = 100000
NEMBED = 128
BATCH = 4096
CTX = 20


_NC = 2
_NS = 16
_NW = _NC * _NS
_BPW = BATCH // _NW
_CHUNK = 4
_NCHUNK = _BPW // _CHUNK
_LANES = NEMBED // 16

_sc_mesh = plsc.VectorSubcoreMesh(core_axis_name="c", subcore_axis_name="s")


@functools.partial(
    pl.kernel,
    mesh=_sc_mesh,
    out_type=jax.ShapeDtypeStruct((BATCH, NEMBED), jnp.float32),
    scratch_types=[
        pltpu.VMEM((_BPW * CTX,), jnp.int32),
        pltpu.VMEM((_CHUNK * CTX, NEMBED), jnp.float32),
        pltpu.VMEM((_BPW, NEMBED), jnp.float32),
        pltpu.SemaphoreType.DMA,
    ],
)
def _gather_mean(idx_hbm, table_hbm, out_hbm, idx_v, rows_v, acc_v, sem):
    wid = lax.axis_index("s") * _NC + lax.axis_index("c")
    base = wid * _BPW
    pltpu.sync_copy(idx_hbm.at[pl.ds(base * CTX, _BPW * CTX)], idx_v)

    def chunk_body(ci, carry):
        pltpu.async_copy(
            table_hbm.at[idx_v.at[pl.ds(ci * _CHUNK * CTX, _CHUNK * CTX)]],
            rows_v,
            sem,
        ).wait()
        for r in range(_CHUNK):
            for d in range(_LANES):
                acc = rows_v[r * CTX, pl.ds(d * 16, 16)]
                for c in range(1, CTX):
                    acc = acc + rows_v[r * CTX + c, pl.ds(d * 16, 16)]
                acc_v[ci * _CHUNK + r, pl.ds(d * 16, 16)] = acc * (1.0 / CTX)
        return carry

    lax.fori_loop(0, _NCHUNK, chunk_body, 0)
    pltpu.sync_copy(acc_v, out_hbm.at[pl.ds(base, _BPW)])


_BB = 1024
_VB = 2048
_NV = (VOCAB + _VB - 1) // _VB
_NI = BATCH // _BB
_NIT = _NV * _NI
_LASTC = VOCAB - (_NV - 1) * _VB
_K = 4


def _mm_body(x_ref, w_ref, b_ref, o_hbm, scratch, tail, sems, tsems):
    v = pl.program_id(0)
    i = pl.program_id(1)
    it = v * _NI + i
    k = lax.rem(it, _K)
    kt = lax.rem(i, 2)

    @pl.when(jnp.logical_and(it >= _K, v < _NV - 1))
    def _wait_prev():
        pltpu.make_async_copy(
            scratch.at[k],
            o_hbm.at[pl.ds(i * _BB, _BB), pl.ds(0, _VB)],
            sems.at[k],
        ).wait()

    @pl.when(jnp.logical_and(v == _NV - 1, i >= 2))
    def _wait_prev_tail():
        pltpu.make_async_copy(
            tail.at[kt],
            o_hbm.at[pl.ds(i * _BB, _BB), pl.ds((_NV - 1) * _VB, _LASTC)],
            tsems.at[kt],
        ).wait()

    x = x_ref[...].astype(jnp.bfloat16)
    w = w_ref[...].astype(jnp.bfloat16)
    acc = lax.dot_general(
        x, w, (((1,), (1,)), ((), ())), preferred_element_type=jnp.float32
    )
    out = acc + b_ref[...]

    @pl.when(v < _NV - 1)
    def _copy_full():
        scratch[k] = out
        pltpu.make_async_copy(
            scratch.at[k],
            o_hbm.at[pl.ds(i * _BB, _BB), pl.ds(v * _VB, _VB)],
            sems.at[k],
        ).start()

    @pl.when(v == _NV - 1)
    def _copy_ragged():
        tail[kt] = out[:, :_LASTC]
        pltpu.make_async_copy(
            tail.at[kt],
            o_hbm.at[pl.ds(i * _BB, _BB), pl.ds((_NV - 1) * _VB, _LASTC)],
            tsems.at[kt],
        ).start()

    @pl.when(it == _NIT - 1)
    def _drain():
        for kk in range(_K):
            pltpu.make_async_copy(
                scratch.at[kk],
                o_hbm.at[pl.ds(0, _BB), pl.ds(0, _VB)],
                sems.at[kk],
            ).wait()
        for kk in range(2):
            pltpu.make_async_copy(
                tail.at[kk],
                o_hbm.at[pl.ds(0, _BB), pl.ds((_NV - 1) * _VB, _LASTC)],
                tsems.at[kk],
            ).wait()


_matmul = pl.pallas_call(
    _mm_body,
    grid=(_NV, _NI),
    in_specs=[
        pl.BlockSpec((_BB, NEMBED), lambda v, i: (i, 0)),
        pl.BlockSpec((_VB, NEMBED), lambda v, i: (v, 0)),
        pl.BlockSpec((1, _VB), lambda v, i: (0, v)),
    ],
    out_specs=pl.BlockSpec(memory_space=pl.ANY),
    out_shape=jax.ShapeDtypeStruct((BATCH, VOCAB), jnp.float32),
    scratch_shapes=[
        pltpu.VMEM((_K, _BB, _VB), jnp.float32),
        pltpu.VMEM((2, _BB, _LASTC), jnp.float32),
        pltpu.SemaphoreType.DMA((_K,)),
        pltpu.SemaphoreType.DMA((2,)),
    ],
    compiler_params=pltpu.CompilerParams(
        dimension_semantics=("arbitrary", "arbitrary"),
    ),
)


def kernel(inp, embed_table, W, b):
    idx = inp.reshape(-1).astype(jnp.int32)
    avg = _gather_mean(idx, embed_table)
    return _matmul(avg, W, b.reshape(1, VOCAB))


# speedup vs baseline: 3.0648x; 3.0254x over previous
import functools

import jax
import jax.numpy as jnp
from jax import lax
from jax.experimental import pallas as pl
from jax.experimental.pallas import tpu as pltpu
from jax.experimental.pallas import tpu_sc as plsc

VOCAB = 100000
NEMBED = 128
BATCH = 4096
CTX = 20


_NC = 2
_NS = 16
_NW = _NC * _NS
_BPW = BATCH // _NW
_CHUNK = 4
_NCHUNK = _BPW // _CHUNK
_LANES = NEMBED // 16

_sc_mesh = plsc.VectorSubcoreMesh(core_axis_name="c", subcore_axis_name="s")


@functools.partial(
    pl.kernel,
    mesh=_sc_mesh,
    out_type=jax.ShapeDtypeStruct((BATCH, NEMBED), jnp.float32),
    scratch_types=[
        pltpu.VMEM((_BPW * CTX,), jnp.int32),
        pltpu.VMEM((_CHUNK * CTX, NEMBED), jnp.float32),
        pltpu.VMEM((_BPW, NEMBED), jnp.float32),
        pltpu.SemaphoreType.DMA,
    ],
)
def _gather_mean(idx_hbm, table_hbm, out_hbm, idx_v, rows_v, acc_v, sem):
    wid = lax.axis_index("s") * _NC + lax.axis_index("c")
    base = wid * _BPW
    pltpu.sync_copy(idx_hbm.at[pl.ds(base * CTX, _BPW * CTX)], idx_v)

    def chunk_body(ci, carry):
        pltpu.async_copy(
            table_hbm.at[idx_v.at[pl.ds(ci * _CHUNK * CTX, _CHUNK * CTX)]],
            rows_v,
            sem,
        ).wait()
        for r in range(_CHUNK):
            for d in range(_LANES):
                acc = rows_v[r * CTX, pl.ds(d * 16, 16)]
                for c in range(1, CTX):
                    acc = acc + rows_v[r * CTX + c, pl.ds(d * 16, 16)]
                acc_v[ci * _CHUNK + r, pl.ds(d * 16, 16)] = acc * (1.0 / CTX)
        return carry

    lax.fori_loop(0, _NCHUNK, chunk_body, 0)
    pltpu.sync_copy(acc_v, out_hbm.at[pl.ds(base, _BPW)])


_BB = 1024
_VB = 2048


def _mm_body(w_ref, x_ref, b_ref, o_ref):
    w = w_ref[...].astype(jnp.bfloat16)
    x = x_ref[...].astype(jnp.bfloat16)
    acc = lax.dot_general(
        w, x, (((1,), (1,)), ((), ())), preferred_element_type=jnp.float32
    )
    o_ref[...] = acc + b_ref[...]


_matmul_t = pl.pallas_call(
    _mm_body,
    grid=(pl.cdiv(VOCAB, _VB), BATCH // _BB),
    in_specs=[
        pl.BlockSpec((_VB, NEMBED), lambda v, i: (v, 0)),
        pl.BlockSpec((_BB, NEMBED), lambda v, i: (i, 0)),
        pl.BlockSpec((_VB, 1), lambda v, i: (v, 0)),
    ],
    out_specs=pl.BlockSpec((_VB, _BB), lambda v, i: (v, i)),
    out_shape=jax.ShapeDtypeStruct((VOCAB, BATCH), jnp.float32),
    compiler_params=pltpu.CompilerParams(
        dimension_semantics=("arbitrary", "arbitrary"),
    ),
)


def kernel(inp, embed_table, W, b):
    idx = inp.reshape(-1).astype(jnp.int32)
    avg = _gather_mean(idx, embed_table)
    return _matmul_t(W, avg, b.reshape(VOCAB, 1)).T


# speedup vs baseline: 3.0907x; 1.0085x over previous
import functools

import jax
import jax.numpy as jnp
from jax import lax
from jax.experimental import pallas as pl
from jax.experimental.pallas import tpu as pltpu
from jax.experimental.pallas import tpu_sc as plsc

VOCAB = 100000
NEMBED = 128
BATCH = 4096
CTX = 20


_NC = 2
_NS = 16
_NW = _NC * _NS
_BPW = BATCH // _NW
_CHUNK = 4
_NCHUNK = _BPW // _CHUNK
_LANES = NEMBED // 16

_sc_mesh = plsc.VectorSubcoreMesh(core_axis_name="c", subcore_axis_name="s")


@functools.partial(
    pl.kernel,
    mesh=_sc_mesh,
    out_type=jax.ShapeDtypeStruct((BATCH, NEMBED), jnp.float32),
    scratch_types=[
        pltpu.VMEM((_BPW * CTX,), jnp.int32),
        pltpu.VMEM((_CHUNK * CTX, NEMBED), jnp.float32),
        pltpu.VMEM((_CHUNK * CTX, NEMBED), jnp.float32),
        pltpu.VMEM((_BPW, NEMBED), jnp.float32),
        pltpu.SemaphoreType.DMA,
        pltpu.SemaphoreType.DMA,
    ],
)
def _gather_mean(idx_hbm, table_hbm, out_hbm, idx_v, rows_a, rows_b, acc_v,
                 sem_a, sem_b):
    wid = lax.axis_index("s") * _NC + lax.axis_index("c")
    base = wid * _BPW
    n_idx = _CHUNK * CTX
    pltpu.sync_copy(idx_hbm.at[pl.ds(base * CTX, _BPW * CTX)], idx_v)

    def _start(ci, buf, sem):
        pltpu.async_copy(table_hbm.at[idx_v.at[pl.ds(ci * n_idx, n_idx)]],
                         buf, sem)

    def _wait(buf, sem):
        pltpu.make_async_copy(table_hbm.at[pl.ds(0, n_idx)], buf, sem).wait()

    def _accum(ci, buf):
        for r in range(_CHUNK):
            for d in range(_LANES):
                acc = buf[r * CTX, pl.ds(d * 16, 16)]
                for c in range(1, CTX):
                    acc = acc + buf[r * CTX + c, pl.ds(d * 16, 16)]
                acc_v[ci * _CHUNK + r, pl.ds(d * 16, 16)] = acc * (1.0 / CTX)

    _start(0, rows_a, sem_a)
    _start(1, rows_b, sem_b)

    def pair_body(j, carry):
        c0 = j * 2
        _wait(rows_a, sem_a)
        _accum(c0, rows_a)

        @pl.when(c0 + 2 < _NCHUNK)
        def _():
            _start(c0 + 2, rows_a, sem_a)

        _wait(rows_b, sem_b)
        _accum(c0 + 1, rows_b)

        @pl.when(c0 + 3 < _NCHUNK)
        def _():
            _start(c0 + 3, rows_b, sem_b)

        return carry

    lax.fori_loop(0, _NCHUNK // 2, pair_body, 0)
    pltpu.sync_copy(acc_v, out_hbm.at[pl.ds(base, _BPW)])


_BB = 1024
_VB = 2048


def _mm_body(w_ref, x_ref, b_ref, o_ref):
    w = w_ref[...].astype(jnp.bfloat16)
    x = x_ref[...].astype(jnp.bfloat16)
    acc = lax.dot_general(
        w, x, (((1,), (1,)), ((), ())), preferred_element_type=jnp.float32
    )
    o_ref[...] = acc + b_ref[...]


_matmul_t = pl.pallas_call(
    _mm_body,
    grid=(pl.cdiv(VOCAB, _VB), BATCH // _BB),
    in_specs=[
        pl.BlockSpec((_VB, NEMBED), lambda v, i: (v, 0)),
        pl.BlockSpec((_BB, NEMBED), lambda v, i: (i, 0)),
        pl.BlockSpec((_VB, 1), lambda v, i: (v, 0)),
    ],
    out_specs=pl.BlockSpec((_VB, _BB), lambda v, i: (v, i)),
    out_shape=jax.ShapeDtypeStruct((VOCAB, BATCH), jnp.float32),
    compiler_params=pltpu.CompilerParams(
        dimension_semantics=("arbitrary", "arbitrary"),
    ),
)


def kernel(inp, embed_table, W, b):
    idx = inp.reshape(-1).astype(jnp.int32)
    avg = _gather_mean(idx, embed_table)
    return _matmul_t(W, avg, b.reshape(VOCAB, 1)).T


# speedup vs baseline: 3.3733x; 1.0914x over previous
import functools

import jax
import jax.numpy as jnp
from jax import lax
from jax.experimental import pallas as pl
from jax.experimental.pallas import tpu as pltpu
from jax.experimental.pallas import tpu_sc as plsc

VOCAB = 100000
NEMBED = 128
BATCH = 4096
CTX = 20


_NC = 2
_NS = 16
_NW = _NC * _NS
_BPW = BATCH // _NW
_CHUNK = 4
_NCHUNK = _BPW // _CHUNK
_LANES = NEMBED // 16

_sc_mesh = plsc.VectorSubcoreMesh(core_axis_name="c", subcore_axis_name="s")


@functools.partial(
    pl.kernel,
    mesh=_sc_mesh,
    out_type=jax.ShapeDtypeStruct((BATCH, NEMBED), jnp.float32),
    scratch_types=[
        pltpu.VMEM((_BPW * CTX,), jnp.int32),
        pltpu.VMEM((_CHUNK * CTX, NEMBED), jnp.float32),
        pltpu.VMEM((_CHUNK * CTX, NEMBED), jnp.float32),
        pltpu.VMEM((_BPW, NEMBED), jnp.float32),
        pltpu.SemaphoreType.DMA,
        pltpu.SemaphoreType.DMA,
    ],
)
def _gather_mean(idx_hbm, table_hbm, out_hbm, idx_v, rows_a, rows_b, acc_v,
                 sem_a, sem_b):
    wid = lax.axis_index("s") * _NC + lax.axis_index("c")
    base = wid * _BPW
    n_idx = _CHUNK * CTX
    pltpu.sync_copy(idx_hbm.at[pl.ds(base * CTX, _BPW * CTX)], idx_v)

    def _start(ci, buf, sem):
        pltpu.async_copy(table_hbm.at[idx_v.at[pl.ds(ci * n_idx, n_idx)]],
                         buf, sem)

    def _wait(buf, sem):
        pltpu.make_async_copy(table_hbm.at[pl.ds(0, n_idx)], buf, sem).wait()

    def _accum(ci, buf):
        for r in range(_CHUNK):
            for d in range(_LANES):
                acc = buf[r * CTX, pl.ds(d * 16, 16)]
                for c in range(1, CTX):
                    acc = acc + buf[r * CTX + c, pl.ds(d * 16, 16)]
                acc_v[ci * _CHUNK + r, pl.ds(d * 16, 16)] = acc * (1.0 / CTX)

    _start(0, rows_a, sem_a)
    _start(1, rows_b, sem_b)

    def pair_body(j, carry):
        c0 = j * 2
        _wait(rows_a, sem_a)
        _accum(c0, rows_a)

        @pl.when(c0 + 2 < _NCHUNK)
        def _():
            _start(c0 + 2, rows_a, sem_a)

        _wait(rows_b, sem_b)
        _accum(c0 + 1, rows_b)

        @pl.when(c0 + 3 < _NCHUNK)
        def _():
            _start(c0 + 3, rows_b, sem_b)

        return carry

    lax.fori_loop(0, _NCHUNK // 2, pair_body, 0)
    pltpu.sync_copy(acc_v, out_hbm.at[pl.ds(base, _BPW)])


_VB = 1024


def _mm_body(w_ref, x_ref, b_ref, o_ref):
    w = w_ref[...].astype(jnp.bfloat16)
    x = x_ref[...].astype(jnp.bfloat16)
    acc = lax.dot_general(
        w, x, (((1,), (1,)), ((), ())), preferred_element_type=jnp.float32
    )
    o_ref[...] = acc + b_ref[...]


_matmul_t = pl.pallas_call(
    _mm_body,
    grid=(pl.cdiv(VOCAB, _VB),),
    in_specs=[
        pl.BlockSpec((_VB, NEMBED), lambda v: (v, 0)),
        pl.BlockSpec((BATCH, NEMBED), lambda v: (0, 0)),
        pl.BlockSpec((_VB, 1), lambda v: (v, 0)),
    ],
    out_specs=pl.BlockSpec((_VB, BATCH), lambda v: (v, 0)),
    out_shape=jax.ShapeDtypeStruct((VOCAB, BATCH), jnp.float32),
    compiler_params=pltpu.CompilerParams(
        dimension_semantics=("arbitrary",),
    ),
)


def kernel(inp, embed_table, W, b):
    idx = inp.reshape(-1).astype(jnp.int32)
    avg = _gather_mean(idx, embed_table)
    return _matmul_t(W, avg, b.reshape(VOCAB, 1)).T
